# baseline (speedup 1.0000x reference)
import functools

import jax
import jax.numpy as jnp
from jax import lax
from jax.experimental import pallas as pl
from jax.experimental.pallas import tpu as pltpu
from jax.experimental.pallas import tpu_sc as plsc

SEQ = 15
N = 3
E = 9
HID = 32
INCH = 512
CLS = 2
LANES = 16


def _adj_sc_body(rsp_hbm, csp_hbm, ewt_hbm, out_hbm, rsp_v, csp_v, ewt_v,
                 out_v):
    f32 = jnp.float32
    wid = lax.axis_index("s") * 2 + lax.axis_index("c")

    @pl.when(wid == 0)
    def _():
        pltpu.sync_copy(rsp_hbm, rsp_v)
        pltpu.sync_copy(csp_hbm, csp_v)
        pltpu.sync_copy(ewt_hbm, ewt_v)
        zero = jnp.zeros((LANES,), f32)
        one = jnp.ones((LANES,), f32)
        W = [zero] * (N * N)
        deg = [zero] * N
        loop = [one] * N
        for e in range(E):
            r_spl = rsp_v[e]
            c_spl = csp_v[e]
            ew_e = ewt_v[e]
            m_self = jnp.where(r_spl == c_spl, one, zero)
            m_row = [jnp.where(r_spl == n, one, zero) for n in range(N)]
            m_col = [jnp.where(c_spl == n, one, zero) for n in range(N)]
            ew_ns = (one - m_self) * ew_e
            for n in range(N):
                deg[n] = deg[n] + m_col[n] * ew_ns
                m = m_self * m_row[n]
                loop[n] = loop[n] * (one - m) + m * ew_e
            for j in range(N * N):
                c, r = j // N, j % N
                W[j] = W[j] + m_col[c] * m_row[r] * ew_ns
        for j in range(N * N):
            out_v[j] = W[j]
        for n in range(N):
            out_v[N * N + n] = deg[n] + loop[n]
            out_v[N * N + N + n] = loop[n]
        out_v[N * N + 2 * N] = zero
        pltpu.sync_copy(out_v, out_hbm)


@functools.cache
def _make_adj_sc():
    return functools.partial(
        pl.kernel,
        out_type=jax.ShapeDtypeStruct((LANES, LANES), jnp.float32),
        mesh=plsc.VectorSubcoreMesh(core_axis_name="c", subcore_axis_name="s"),
        scratch_types=[
            pltpu.VMEM((LANES, LANES), jnp.int32),
            pltpu.VMEM((LANES, LANES), jnp.int32),
            pltpu.VMEM((LANES, LANES), jnp.float32),
            pltpu.VMEM((LANES, LANES), jnp.float32),
        ],
    )(_adj_sc_body)


def _gcn_tc_kernel(x_ref, adj_ref, w1_ref, b1_ref, w2_ref, b2_ref,
                   wl_ref, bl_ref, out_ref):
    f32 = jnp.float32
    adj = adj_ref[...]
    W_T = adj[0:N * N, 0:SEQ]
    deg_T = adj[N * N:N * N + N, 0:SEQ]
    loop_T = adj[N * N + N:N * N + 2 * N, 0:SEQ]
    dinv_T = jnp.where(deg_T > 0, lax.rsqrt(deg_T), jnp.zeros_like(deg_T))

    dot_nn = lambda a, b: lax.dot_general(a, b, (((1,), (1,)), ((), ())),
                                          preferred_element_type=f32)
    jj = lax.broadcasted_iota(jnp.int32, (N * N, N), 0)
    nn = lax.broadcasted_iota(jnp.int32, (N * N, N), 1)
    Cmap = ((jj // N) == nn).astype(f32)
    Rmap = ((jj % N) == nn).astype(f32)
    mm = lambda a, b: jnp.dot(a, b, preferred_element_type=f32)
    A_T = (mm(Cmap, dinv_T) * mm(Rmap, dinv_T)
           * (W_T + mm(Cmap * Rmap, loop_T)))
    eye9 = (lax.broadcasted_iota(jnp.int32, (N * N, N * N), 0)
            == lax.broadcasted_iota(jnp.int32, (N * N, N * N), 1)).astype(f32)
    A = lax.dot_general(A_T, eye9, (((0,), (0,)), ((), ())),
                        preferred_element_type=f32)

    H_t = dot_nn(x_ref[...], w1_ref[...])
    q_s = lax.broadcasted_iota(jnp.int32, (N * SEQ, N * SEQ), 0)
    k_l = lax.broadcasted_iota(jnp.int32, (N * SEQ, N * SEQ), 1)
    P = (k_l == N * (q_s % SEQ) + q_s // SEQ).astype(f32)
    H = jnp.dot(P, H_t, preferred_element_type=f32)

    b1 = b1_ref[...].reshape(1, HID)
    h1 = []
    for c in range(N):
        acc = jnp.broadcast_to(b1, (SEQ, HID))
        for r in range(N):
            acc = acc + A[:, N * c + r:N * c + r + 1] * H[SEQ * r:SEQ * (r + 1), :]
        h1.append(jnp.maximum(acc, 0.0))

    G = dot_nn(jnp.concatenate(h1, axis=0), w2_ref[...])
    b2 = b2_ref[...].reshape(1, HID)
    wl = wl_ref[...]
    inv_seq = f32(1.0 / SEQ)
    y = jnp.broadcast_to(bl_ref[...].reshape(1, CLS), (1, CLS))
    for c in range(N):
        acc = jnp.broadcast_to(b2, (SEQ, HID))
        for r in range(N):
            acc = acc + A[:, N * c + r:N * c + r + 1] * G[SEQ * r:SEQ * (r + 1), :]
        pooled = jnp.sum(acc, axis=0, keepdims=True) * inv_seq
        y = y + dot_nn(pooled, wl[:, HID * c:HID * (c + 1)])
    out_ref[...] = y


def kernel(x, edge_index, edge_weight, W1, b1, W2, b2, Wl, bl):
    rsp = jnp.zeros((LANES, LANES), jnp.int32).at[:E, :].set(
        jnp.broadcast_to(edge_index[0][:, None], (E, LANES)))
    csp = jnp.zeros((LANES, LANES), jnp.int32).at[:E, :].set(
        jnp.broadcast_to(edge_index[1][:, None], (E, LANES)))
    ewt_pad = jnp.zeros((LANES, LANES), jnp.float32).at[:E, :SEQ].set(
        edge_weight.T)
    adj = _make_adj_sc()(rsp, csp, ewt_pad)
    out = pl.pallas_call(
        _gcn_tc_kernel,
        out_shape=jax.ShapeDtypeStruct((1, CLS), jnp.float32),
    )(x.reshape(N * SEQ, INCH), adj, W1, b1, W2, b2, Wl, bl)
    return out.reshape(CLS)
